# baseline (speedup 1.0000x reference)
import functools

import jax
import jax.numpy as jnp
from jax.experimental import pallas as pl
from jax.experimental.pallas import tpu as pltpu


def _cc_kernel(left_ref, right_ref, win_ref, br_ref,
               wck_ref, out_ref, *, L, NB, T):
    f32 = jnp.float32
    bf16 = jnp.bfloat16
    R = NB * L
    NSPLIT = 1
    TC = T // NSPLIT

    def chunk(c):
        cols = pl.ds(c * TC, TC)
        x = jnp.concatenate([left_ref[:, cols], right_ref[:, cols]], axis=0)
        ps = jnp.dot(win_ref[...], x,
                     preferred_element_type=f32)
        sf = ps[0:R, :] + br_ref[...]
        sfb = sf.astype(bf16)
        prodall = jnp.concatenate(
            [sfb]
            + [ps[(j + 1) * R:(j + 2) * R, :].astype(bf16) * sfb
               for j in range(L)],
            axis=0)
        return jnp.dot(wck_ref[...], prodall,
                       preferred_element_type=f32)

    yall = jnp.concatenate([chunk(c) for c in range(NSPLIT)], axis=1)
    y0 = yall[0:R, :]
    y1 = yall[R:2 * R, :]
    y2 = yall[2 * R:3 * R, :]

    t = jax.lax.broadcasted_iota(jnp.int32, (1, T), 1)
    not_first = (t != 0).astype(f32)
    not_last = (t != T - 1).astype(f32)
    out_ref[...] = (y1
                    + not_first * pltpu.roll(y0, 1, axis=1)
                    + not_last * pltpu.roll(y2, T - 1, axis=1))


def _pick_nb(B, L, T):
    best = 1
    for nb in range(1, B + 1):
        if B % nb:
            continue
        rows = nb * L
        if rows > 128 or rows % 8:
            continue
        if nb * L * T * 4 > 2 * 1024 * 1024:
            continue
        best = nb
    return best


def kernel(left, right, wl, bl, wr, br, wconv):
    B, L, T = left.shape
    f32 = jnp.float32
    NB = _pick_nb(B, L, T)

    eye = jnp.eye(NB, dtype=f32)
    wl_f = wl.astype(f32)
    bl_f = bl.astype(f32)
    wc_f = wconv.astype(f32)
    ones_l = jnp.ones((L, 1), f32)
    R = NB * L
    WLB = jnp.concatenate(
        [jnp.kron(eye, ones_l @ wl_f[j:j + 1, :]) for j in range(L)],
        axis=0)
    WR = jnp.kron(eye, wr.astype(f32)[::-1, :])
    zeros_r = jnp.zeros((L * R, R), f32)
    WIN = jnp.block([[jnp.zeros((R, R), f32), WR], [WLB, zeros_r]])
    BR = jnp.tile(br.astype(f32)[::-1], NB).reshape(R, 1)

    def conv_slabs(k):
        wck = wc_f[:, :, k]
        ck = sum(bl_f[j] * wck[:, j:j + L] for j in range(L))
        blocks = [jnp.kron(eye, ck)]
        blocks += [jnp.kron(eye, wck[:, j:j + L]) for j in range(L)]
        return jnp.concatenate(blocks, axis=1)

    bf16 = jnp.bfloat16
    WCK = jnp.concatenate([conv_slabs(k) for k in range(3)],
                          axis=0).astype(bf16)

    left2 = left.astype(f32).reshape(B * L, T)
    right2 = right.astype(f32).reshape(B * L, T)

    rows = NB * L
    io = pl.BlockSpec((rows, T), lambda i: (i, 0))
    cst = lambda shape: pl.BlockSpec(shape, lambda i: (0, 0))

    out2 = pl.pallas_call(
        functools.partial(_cc_kernel, L=L, NB=NB, T=T),
        out_shape=jax.ShapeDtypeStruct((B * L, T), f32),
        grid=(B // NB,),
        in_specs=[io, io,
                  cst(((L + 1) * rows, 2 * rows)), cst((rows, 1)),
                  cst((3 * rows, (L + 1) * rows))],
        out_specs=io,
        compiler_params=pltpu.CompilerParams(
            dimension_semantics=("parallel",),
            vmem_limit_bytes=64 * 1024 * 1024),
    )(left2, right2, WIN, BR, WCK)
    return out2.reshape(B, L, T)


# speedup vs baseline: 1.1926x; 1.1926x over previous
import functools

import jax
import jax.numpy as jnp
from jax.experimental import pallas as pl
from jax.experimental.pallas import tpu as pltpu


def _cc_kernel(left_ref, right_ref, win_ref, br_ref,
               wck_ref, out_ref, *, L, NB, T):
    f32 = jnp.float32
    bf16 = jnp.bfloat16
    R = NB * L
    NSPLIT = 1
    TC = T // NSPLIT

    def chunk(c):
        cols = pl.ds(c * TC, TC)
        x = jnp.concatenate([left_ref[:, cols], right_ref[:, cols]], axis=0)
        ps = jnp.dot(win_ref[...], x,
                     preferred_element_type=f32)
        sf = ps[0:R, :] + br_ref[...]
        sfb = sf.astype(bf16)
        prodall = jnp.concatenate(
            [sfb]
            + [ps[(j + 1) * R:(j + 2) * R, :].astype(bf16) * sfb
               for j in range(L)],
            axis=0)
        return jnp.dot(wck_ref[...], prodall,
                       preferred_element_type=f32)

    yall = jnp.concatenate([chunk(c) for c in range(NSPLIT)], axis=1)
    y0 = yall[0:R, :]
    y1 = yall[R:2 * R, :]
    y2 = yall[2 * R:3 * R, :]

    t = jax.lax.broadcasted_iota(jnp.int32, (1, T), 1)
    not_first = (t != 0).astype(f32)
    not_last = (t != T - 1).astype(f32)
    out_ref[...] = (y1
                    + not_first * pltpu.roll(y0, 1, axis=1)
                    + not_last * pltpu.roll(y2, T - 1, axis=1))


def _pick_nb(B, L, T):
    best = 1
    for nb in range(1, B + 1):
        if B % nb:
            continue
        rows = nb * L
        if rows > 128 or rows % 8:
            continue
        if nb * L * T * 4 > 2 * 1024 * 1024:
            continue
        best = nb
    return best


def kernel(left, right, wl, bl, wr, br, wconv):
    B, L, T = left.shape
    f32 = jnp.float32
    NB = _pick_nb(B, L, T)

    bf16 = jnp.bfloat16
    eye = jnp.eye(NB, dtype=f32)
    wl_f = wl.astype(f32)
    bl_f = bl.astype(f32)
    wc_f = wconv.astype(f32)
    R = NB * L

    m_sf = jnp.stack([jnp.zeros((L, L), f32), wr.astype(f32)[::-1, :]],
                     axis=1)
    side0 = jnp.array([1.0, 0.0], f32)
    m_p = (jnp.broadcast_to(wl_f[:, None, None, :], (L, L, 2, L))
           * side0[None, None, :, None])
    M = jnp.concatenate([m_sf[None], m_p], axis=0)
    WIN = (eye[None, :, None, None, :, None]
           * M[:, None, :, :, None, :]).reshape((L + 1) * R, 2 * R)
    BR = jnp.tile(br.astype(f32)[::-1], NB).reshape(R, 1)

    idx = jnp.arange(L)[:, None] + jnp.arange(L)[None, :]
    win4 = wc_f[:, idx, :]
    ck = jnp.einsum('j,cjmk->kcm', bl_f, win4)[:, :, None, :]
    C = jnp.concatenate([ck, win4.transpose(3, 0, 1, 2)], axis=2)
    WCK = (eye[None, :, None, None, :, None]
           * C[:, None, :, :, None, :]).reshape(3 * R, (L + 1) * R).astype(bf16)

    left2 = left.astype(f32).reshape(B * L, T)
    right2 = right.astype(f32).reshape(B * L, T)

    rows = NB * L
    io = pl.BlockSpec((rows, T), lambda i: (i, 0))
    cst = lambda shape: pl.BlockSpec(shape, lambda i: (0, 0))

    out2 = pl.pallas_call(
        functools.partial(_cc_kernel, L=L, NB=NB, T=T),
        out_shape=jax.ShapeDtypeStruct((B * L, T), f32),
        grid=(B // NB,),
        in_specs=[io, io,
                  cst(((L + 1) * rows, 2 * rows)), cst((rows, 1)),
                  cst((3 * rows, (L + 1) * rows))],
        out_specs=io,
        compiler_params=pltpu.CompilerParams(
            dimension_semantics=("parallel",),
            vmem_limit_bytes=64 * 1024 * 1024),
    )(left2, right2, WIN, BR, WCK)
    return out2.reshape(B, L, T)
